# baseline (speedup 1.0000x reference)
import functools

import jax
import jax.numpy as jnp
from jax import lax
from jax.experimental import pallas as pl
from jax.experimental.pallas import tpu as pltpu
from jax.experimental.pallas import tpu_sc as plsc

S, H = 2048, 1024
E, NG, TOPK = 8, 4, 2
I = 512
SH_I = 1024
RSF = 2.5
NEG = -1e30

BLK = 256
NBMAX = (TOPK * S) // BLK + E
NBPAD = 32
P = NBMAX * BLK
A = TOPK * S
SENT = 2 * A
OUT12_R = A + BLK
NWORK = 32
SLOTS_W = P // NWORK


def _gate_body(x_ref, gk_ref, eb_ref, eid_ref, wk_ref):
    x = x_ref[...]
    logits = jnp.dot(x, gk_ref[...], preferred_element_type=jnp.float32)
    scores = jax.nn.sigmoid(logits)
    sfc = scores + eb_ref[...]
    lane = lax.broadcasted_iota(jnp.int32, (S, E), 1)
    grp = lane // (E // NG)
    gsum = jnp.zeros_like(sfc)
    for g in range(NG):
        sg = jnp.sum(jnp.where(grp == g, sfc, 0.0), axis=1, keepdims=True)
        gsum = gsum + jnp.where(grp == g, sg, 0.0)
    m1 = jnp.max(gsum, axis=1, keepdims=True)
    g1 = jnp.min(jnp.where(gsum == m1, grp, E), axis=1, keepdims=True)
    p2 = jnp.where(grp == g1, NEG, gsum)
    m2 = jnp.max(p2, axis=1, keepdims=True)
    g2 = jnp.min(jnp.where(p2 == m2, grp, E), axis=1, keepdims=True)
    gmask = (grp == g1) | (grp == g2)
    ms = jnp.where(gmask, sfc, 0.0)
    t1 = jnp.max(ms, axis=1, keepdims=True)
    i1 = jnp.min(jnp.where(ms == t1, lane, E), axis=1, keepdims=True)
    ms2 = jnp.where(lane == i1, NEG, ms)
    t2 = jnp.max(ms2, axis=1, keepdims=True)
    i2 = jnp.min(jnp.where(ms2 == t2, lane, E), axis=1, keepdims=True)
    denom = t1 + t2 + 1e-20
    w1 = t1 * (RSF / denom)
    w2 = t2 * (RSF / denom)
    eid_ref[...] = jnp.concatenate([i1, i2], axis=1)
    wk_ref[...] = jnp.concatenate([w1, w2], axis=1)


def _gate(x, gate_kernel, e_bias):
    const = lambda: (0, 0)
    return pl.pallas_call(
        _gate_body,
        in_specs=[
            pl.BlockSpec((S, H), const),
            pl.BlockSpec((H, E), const),
            pl.BlockSpec((1, E), const),
        ],
        out_specs=[
            pl.BlockSpec((S, TOPK), const),
            pl.BlockSpec((S, TOPK), const),
        ],
        out_shape=[
            jax.ShapeDtypeStruct((S, TOPK), jnp.int32),
            jax.ShapeDtypeStruct((S, TOPK), jnp.float32),
        ],
    )(x, gate_kernel, e_bias)


def _dispatch_body(eid_hbm, flat_hbm, bex_hbm, eid_v, myflat_v, bexbuf_v):
    c = lax.axis_index("c")
    s = lax.axis_index("s")
    active = jnp.logical_and(c == 0, s < E)
    iota = lax.broadcasted_iota(jnp.int32, (16,), 0)

    @pl.when(active)
    def _work():
        w = s
        pltpu.sync_copy(eid_hbm, eid_v)

        def prefill(j, carry):
            myflat_v[pl.ds(j * 16, 16)] = jnp.full((16,), SENT, jnp.int32)
            return carry

        lax.fori_loop(0, A // 16, prefill, jnp.int32(0))

        zeros = jnp.zeros((16,), jnp.int32)

        def body(j, carry):
            cnt, cnts_all = carry
            v = eid_v[pl.ds(j * 16, 16)]
            m = v == w
            flat = iota + j * 16
            ones = jnp.where(m, jnp.ones((16,), jnp.int32), zeros)
            r = plsc.cumsum(ones)
            if DBG_D_SCATTER:
                slots = cnt + r - 1
                plsc.store_scatter(myflat_v, [slots], flat, mask=m)
            for e in range(E):
                pc = plsc.all_reduce_population_count(v == e)
                cnts_all = cnts_all + jnp.where(iota == e, pc, zeros)
            return cnt + jnp.max(r), cnts_all

        cnt, cnts_all = lax.fori_loop(0, A // 16, body,
                                      (jnp.int32(0), zeros))
        off = jnp.int32(0)
        for e in range(E):
            ce = jnp.max(jnp.where(iota == e, cnts_all, zeros))
            nb = (ce + BLK - 1) // BLK
            off = off + jnp.where(e < w, nb * BLK, 0)
        my_nb = (cnt + BLK - 1) // BLK
        if DBG_D_EMIT:
            for j in range(E):
                @pl.when(j < my_nb)
                def _cp():
                    dst = pl.multiple_of(off + j * BLK, BLK)
                    pltpu.sync_copy(myflat_v.at[pl.ds(j * BLK, BLK)],
                                    flat_hbm.at[pl.ds(dst, BLK)])

        @pl.when(w == 0)
        def _bex():
            for vb in range(NBPAD // 16):
                bidx = iota + vb * 16
                be = jnp.full((16,), E, jnp.int32)
                accblk = jnp.int32(0)
                for e in range(E):
                    ce = jnp.max(jnp.where(iota == e, cnts_all, zeros))
                    nb = (ce + BLK - 1) // BLK
                    be = jnp.where((bidx >= accblk) & (bidx < accblk + nb),
                                   e, be)
                    accblk = accblk + nb
                bexbuf_v[pl.ds(vb * 16, 16)] = be
            pltpu.sync_copy(bexbuf_v, bex_hbm)


def _dispatch(eid_flat):
    mesh = plsc.VectorSubcoreMesh(core_axis_name="c", subcore_axis_name="s")
    k = functools.partial(
        pl.kernel,
        out_type=[
            jax.ShapeDtypeStruct((P,), jnp.int32),
            jax.ShapeDtypeStruct((NBPAD,), jnp.int32),
        ],
        mesh=mesh,
        compiler_params=pltpu.CompilerParams(needs_layout_passes=False),
        scratch_types=[
            pltpu.VMEM((A,), jnp.int32),
            pltpu.VMEM((A,), jnp.int32),
            pltpu.VMEM((NBPAD,), jnp.int32),
        ],
    )(_dispatch_body)
    return k(eid_flat)


CH = 48
NCH = SLOTS_W // CH


def _gather_body(x_hbm, flat_hbm, xs_hbm, flat_v,
                 i0, i1, i2, i3, b0, b1, s0, s1, o0, o1):
    c = lax.axis_index("c")
    s = lax.axis_index("s")
    wid = s * 2 + c
    base = wid * SLOTS_W
    idxs = [i0, i1, i2, i3]
    bufs = [b0, b1]
    isems = [s0, s1]
    osems = [o0, o1]
    pltpu.sync_copy(flat_hbm.at[pl.ds(base, SLOTS_W)], flat_v)
    for ch in range(NCH):
        for j in range(CH // 16):
            fvec = flat_v[pl.ds(ch * CH + j * 16, 16)]
            idxs[ch][pl.ds(j * 16, 16)] = jnp.clip(fvec, 0, A - 1) >> 1
    outh = [None, None]
    for ch in range(NCH):
        b = ch & 1
        if outh[b] is not None:
            outh[b].wait()
        pltpu.async_copy(x_hbm.at[idxs[ch]], bufs[b], isems[b]).wait()
        outh[b] = pltpu.async_copy(
            bufs[b], xs_hbm.at[pl.ds(base + ch * CH, CH)], osems[b])
    outh[0].wait()
    outh[1].wait()


def _gather(x, flat_sorted):
    mesh = plsc.VectorSubcoreMesh(core_axis_name="c", subcore_axis_name="s")
    k = functools.partial(
        pl.kernel,
        out_type=jax.ShapeDtypeStruct((P, H), jnp.float32),
        mesh=mesh,
        compiler_params=pltpu.CompilerParams(needs_layout_passes=False),
        scratch_types=[
            pltpu.VMEM((SLOTS_W,), jnp.int32),
            pltpu.VMEM((CH,), jnp.int32),
            pltpu.VMEM((CH,), jnp.int32),
            pltpu.VMEM((CH,), jnp.int32),
            pltpu.VMEM((CH,), jnp.int32),
            pltpu.VMEM((CH, H), jnp.float32),
            pltpu.VMEM((CH, H), jnp.float32),
            pltpu.SemaphoreType.DMA,
            pltpu.SemaphoreType.DMA,
            pltpu.SemaphoreType.DMA,
            pltpu.SemaphoreType.DMA,
        ],
    )(_gather_body)
    return k(x, flat_sorted)


def _gmm_body(bex_ref, xs_ref, eg_ref, eu_ref, ed_ref, ys_ref):
    b = pl.program_id(0)

    @pl.when(bex_ref[b] != E)
    def _():
        xb = xs_ref[...]
        g = jnp.dot(xb, eg_ref[0], preferred_element_type=jnp.float32)
        u = jnp.dot(xb, eu_ref[0], preferred_element_type=jnp.float32)
        a = jax.nn.silu(g) * u
        ys_ref[...] = jnp.dot(a, ed_ref[0], preferred_element_type=jnp.float32)


def _gmm(xs, bex, expert_gate, expert_up, expert_down):
    wmap = lambda b, bex_ref: (jnp.minimum(bex_ref[b], E - 1), 0, 0)
    return pl.pallas_call(
        _gmm_body,
        grid_spec=pltpu.PrefetchScalarGridSpec(
            num_scalar_prefetch=1,
            grid=(NBMAX,),
            in_specs=[
                pl.BlockSpec((BLK, H), lambda b, bex_ref: (b, 0)),
                pl.BlockSpec((1, H, I), wmap),
                pl.BlockSpec((1, H, I), wmap),
                pl.BlockSpec((1, I, H), wmap),
            ],
            out_specs=pl.BlockSpec((BLK, H), lambda b, bex_ref: (b, 0)),
        ),
        out_shape=jax.ShapeDtypeStruct((P, H), jnp.float32),
        compiler_params=pltpu.CompilerParams(
            dimension_semantics=("arbitrary",),
        ),
    )(bex, xs, expert_gate, expert_up, expert_down)


JUNK = A


def _unpermute_body(ys_hbm, flat_hbm, bex_hbm, out12_hbm,
                    flat_v, bex_v, i0, i1, i2, i3, b0, b1, s0, s1, o0, o1):
    c = lax.axis_index("c")
    s = lax.axis_index("s")
    wid = s * 2 + c
    base = wid * SLOTS_W
    iota = lax.broadcasted_iota(jnp.int32, (16,), 0)
    idxs = [i0, i1, i2, i3]
    bufs = [b0, b1]
    isems = [s0, s1]
    osems = [o0, o1]
    pltpu.sync_copy(flat_hbm.at[pl.ds(base, SLOTS_W)], flat_v)
    pltpu.sync_copy(bex_hbm, bex_v)
    junk = jnp.full((16,), JUNK, jnp.int32)
    for ch in range(NCH):
        for j in range(CH // 16):
            sl = ch * CH + j * 16
            bi = (base + sl) // BLK
            bvec = bex_v[pl.ds((bi // 16) * 16, 16)]
            be = jnp.max(jnp.where(iota == (bi % 16), bvec, -1))
            fvec = flat_v[pl.ds(sl, 16)]
            d = jnp.clip((fvec >> 1) + S * (fvec & 1), 0, OUT12_R - 1)
            valid = jnp.full((16,), be, jnp.int32) != E
            idxs[ch][pl.ds(j * 16, 16)] = jnp.where(valid, d, junk)
    outh = [None, None]
    for ch in range(NCH):
        b = ch & 1
        if outh[b] is not None:
            outh[b].wait()
        pltpu.async_copy(ys_hbm.at[pl.ds(base + ch * CH, CH)],
                         bufs[b], isems[b]).wait()
        outh[b] = pltpu.async_copy(bufs[b], out12_hbm.at[idxs[ch]], osems[b])
    outh[0].wait()
    outh[1].wait()


def _unpermute(ys, flat_sorted, bex):
    mesh = plsc.VectorSubcoreMesh(core_axis_name="c", subcore_axis_name="s")
    k = functools.partial(
        pl.kernel,
        out_type=jax.ShapeDtypeStruct((OUT12_R, H), jnp.float32),
        mesh=mesh,
        compiler_params=pltpu.CompilerParams(needs_layout_passes=False),
        scratch_types=[
            pltpu.VMEM((SLOTS_W,), jnp.int32),
            pltpu.VMEM((NBPAD,), jnp.int32),
            pltpu.VMEM((CH,), jnp.int32),
            pltpu.VMEM((CH,), jnp.int32),
            pltpu.VMEM((CH,), jnp.int32),
            pltpu.VMEM((CH,), jnp.int32),
            pltpu.VMEM((CH, H), jnp.float32),
            pltpu.VMEM((CH, H), jnp.float32),
            pltpu.SemaphoreType.DMA,
            pltpu.SemaphoreType.DMA,
            pltpu.SemaphoreType.DMA,
            pltpu.SemaphoreType.DMA,
        ],
    )(_unpermute_body)
    return k(ys, flat_sorted, bex)


ST = S // 2


def _final_body(x_ref, y1_ref, y2_ref, wk_ref, shg_ref, shu_ref, shd_ref,
                out_ref):
    x = x_ref[...]
    g = jnp.dot(x, shg_ref[...], preferred_element_type=jnp.float32)
    u = jnp.dot(x, shu_ref[...], preferred_element_type=jnp.float32)
    a = jax.nn.silu(g) * u
    sh = jnp.dot(a, shd_ref[...], preferred_element_type=jnp.float32)
    w1 = wk_ref[:, 0:1]
    w2 = wk_ref[:, 1:2]
    out_ref[...] = w1 * y1_ref[...] + w2 * y2_ref[...] + sh


def _final(x, out12, wk, sh_gate, sh_up, sh_down):
    const = lambda t: (0, 0)
    return pl.pallas_call(
        _final_body,
        grid=(S // ST,),
        in_specs=[
            pl.BlockSpec((ST, H), lambda t: (t, 0)),
            pl.BlockSpec((ST, H), lambda t: (t, 0)),
            pl.BlockSpec((ST, H), lambda t: (t + S // ST, 0)),
            pl.BlockSpec((ST, TOPK), lambda t: (t, 0)),
            pl.BlockSpec((H, SH_I), const),
            pl.BlockSpec((H, SH_I), const),
            pl.BlockSpec((SH_I, H), const),
        ],
        out_specs=pl.BlockSpec((ST, H), lambda t: (t, 0)),
        out_shape=jax.ShapeDtypeStruct((S, H), jnp.float32),
        compiler_params=pltpu.CompilerParams(
            dimension_semantics=("arbitrary",),
        ),
    )(x, out12, out12, wk, sh_gate, sh_up, sh_down)


DBG_STAGE = 3
DBG_D_SCATTER = True
DBG_D_EMIT = True


@jax.jit
def _moe(x, gate_kernel, e_bias, expert_gate, expert_up, expert_down,
         sh_gate, sh_up, sh_down):
    eid, wk = _gate(x, gate_kernel, e_bias)
    eidf = eid.reshape(A)
    if DBG_STAGE >= 1:
        flat_sc, bex = _dispatch(eidf)
        if DBG_D_SCATTER and DBG_D_EMIT:
            flat_sorted = flat_sc
        else:
            flat_sorted = None
    if DBG_STAGE < 1 or flat_sorted is None:
        order = jnp.argsort(eidf, stable=True)
        se = eidf[order]
        cnts = jnp.bincount(eidf, length=E)
        nb = (cnts + BLK - 1) // BLK
        offs = jnp.concatenate([jnp.zeros(1, jnp.int32),
                                jnp.cumsum(nb * BLK)[:-1].astype(jnp.int32)])
        rank = jnp.arange(A) - jnp.cumsum(
            jnp.concatenate([jnp.zeros(1, jnp.int32), cnts[:-1]]))[se]
        slots = offs[se] + rank
        flat_sorted = jnp.full((P,), SENT, jnp.int32).at[slots].set(
            order.astype(jnp.int32))
        bexv = jnp.full((NBPAD,), E, jnp.int32)
        bids = jnp.arange(NBPAD)
        accb = jnp.concatenate([jnp.zeros(1, jnp.int32),
                                jnp.cumsum(nb).astype(jnp.int32)])
        for e in range(E):
            bexv = jnp.where((bids >= accb[e]) & (bids < accb[e + 1]), e, bexv)
        if DBG_STAGE < 1:
            bex = bexv
    if DBG_STAGE >= 2:
        xs = _gather(x, flat_sorted)
    else:
        xs = x[jnp.clip(flat_sorted, 0, A - 1) >> 1]
    ys = _gmm(xs, bex, expert_gate, expert_up, expert_down)
    if DBG_STAGE >= 3:
        out12 = _unpermute(ys, flat_sorted, bex)
    else:
        validslot = jnp.repeat(bex[:NBMAX] != E, BLK)
        d = (jnp.clip(flat_sorted, 0, SENT) >> 1) + S * (flat_sorted & 1)
        d = jnp.where(validslot, d, A)
        out12 = jnp.zeros((OUT12_R, H), jnp.float32).at[d].set(ys)
    return _final(x, out12, wk, sh_gate, sh_up, sh_down)


def kernel(hidden_states, gate_kernel, e_bias, expert_gate, expert_up,
           expert_down, sh_gate, sh_up, sh_down):
    b, s, h = hidden_states.shape
    x = hidden_states.reshape(s, h)
    y = _moe(x, gate_kernel, e_bias.reshape(1, E), expert_gate, expert_up,
             expert_down, sh_gate, sh_up, sh_down)
    return y.reshape(b, s, h)


# speedup vs baseline: 3.5615x; 3.5615x over previous
import functools

import jax
import jax.numpy as jnp
from jax.experimental import pallas as pl
from jax.experimental.pallas import tpu as pltpu

S, H = 2048, 1024
E, NG, TOPK = 8, 4, 2
I = 512
SH_I = 1024
RSF = 2.5
NEG = -1e30


def _moe_body(x_ref, gk_ref, eb_ref, eg_ref, eu_ref, ed_ref,
              shg_ref, shu_ref, shd_ref, out_ref, w_ref, acc_ref):
    e = pl.program_id(0)

    @pl.when(e == 0)
    def _gate():
        x = x_ref[...]
        logits = jnp.dot(x, gk_ref[...], preferred_element_type=jnp.float32)
        scores = jax.nn.sigmoid(logits)
        sfc = scores + eb_ref[...]
        lane = jax.lax.broadcasted_iota(jnp.int32, (S, E), 1)
        grp = lane // (E // NG)
        pairsum = jnp.zeros_like(sfc)
        for g in range(NG):
            sg = jnp.sum(jnp.where(grp == g, sfc, 0.0), axis=1, keepdims=True)
            pairsum = pairsum + jnp.where(grp == g, sg, 0.0)
        m1 = jnp.max(pairsum, axis=1, keepdims=True)
        g1 = jnp.min(jnp.where(pairsum == m1, grp, E), axis=1, keepdims=True)
        p2 = jnp.where(grp == g1, NEG, pairsum)
        m2 = jnp.max(p2, axis=1, keepdims=True)
        g2 = jnp.min(jnp.where(p2 == m2, grp, E), axis=1, keepdims=True)
        gmask = (grp == g1) | (grp == g2)
        ms = jnp.where(gmask, sfc, 0.0)
        t1 = jnp.max(ms, axis=1, keepdims=True)
        i1 = jnp.min(jnp.where(ms == t1, lane, E), axis=1, keepdims=True)
        ms2 = jnp.where(lane == i1, NEG, ms)
        t2 = jnp.max(ms2, axis=1, keepdims=True)
        i2 = jnp.min(jnp.where(ms2 == t2, lane, E), axis=1, keepdims=True)
        denom = t1 + t2 + 1e-20
        w_ref[...] = (jnp.where(lane == i1, t1, 0.0)
                      + jnp.where(lane == i2, t2, 0.0)) * (RSF / denom)

    @pl.when(e < E)
    def _expert():
        xb = x_ref[...]
        lane = jax.lax.broadcasted_iota(jnp.int32, (S, E), 1)
        w = jnp.sum(jnp.where(lane == e, w_ref[...], 0.0), axis=1, keepdims=True)
        g = jnp.dot(xb, eg_ref[0],
                    preferred_element_type=jnp.float32).astype(jnp.bfloat16)
        u = jnp.dot(xb, eu_ref[0],
                    preferred_element_type=jnp.float32).astype(jnp.bfloat16)
        a = jax.nn.silu(g) * u
        y = jnp.dot(a, ed_ref[0].astype(jnp.bfloat16),
                    preferred_element_type=jnp.float32)

        @pl.when(e == 0)
        def _():
            acc_ref[...] = w * y

        @pl.when(e > 0)
        def _():
            acc_ref[...] = acc_ref[...] + w * y

    @pl.when(e == E)
    def _shared():
        xb = x_ref[...]
        g = jnp.dot(xb, shg_ref[...],
                    preferred_element_type=jnp.float32).astype(jnp.bfloat16)
        u = jnp.dot(xb, shu_ref[...],
                    preferred_element_type=jnp.float32).astype(jnp.bfloat16)
        a = jax.nn.silu(g) * u
        y = jnp.dot(a, shd_ref[...].astype(jnp.bfloat16),
                    preferred_element_type=jnp.float32)
        out_ref[...] = acc_ref[...] + y


@jax.jit
def _moe(x, gate_kernel, e_bias, expert_gate, expert_up, expert_down,
         sh_gate, sh_up, sh_down):
    const = lambda e: (0, 0)
    return pl.pallas_call(
        _moe_body,
        grid=(E + 1,),
        in_specs=[
            pl.BlockSpec((S, H), const),
            pl.BlockSpec((H, E), const),
            pl.BlockSpec((1, E), const),
            pl.BlockSpec((1, H, I), lambda e: (jnp.minimum(e, E - 1), 0, 0)),
            pl.BlockSpec((1, H, I), lambda e: (jnp.minimum(e, E - 1), 0, 0)),
            pl.BlockSpec((1, I, H), lambda e: (jnp.minimum(e, E - 1), 0, 0)),
            pl.BlockSpec((H, SH_I), const),
            pl.BlockSpec((H, SH_I), const),
            pl.BlockSpec((SH_I, H), const),
        ],
        out_specs=pl.BlockSpec((S, H), const),
        out_shape=jax.ShapeDtypeStruct((S, H), jnp.float32),
        scratch_shapes=[
            pltpu.VMEM((S, E), jnp.float32),
            pltpu.VMEM((S, H), jnp.float32),
        ],
        compiler_params=pltpu.CompilerParams(
            dimension_semantics=("arbitrary",),
            vmem_limit_bytes=100 * 1024 * 1024,
        ),
    )(x, gate_kernel, e_bias, expert_gate, expert_up, expert_down,
      sh_gate, sh_up, sh_down)


def kernel(hidden_states, gate_kernel, e_bias, expert_gate, expert_up,
           expert_down, sh_gate, sh_up, sh_down):
    b, s, h = hidden_states.shape
    x = hidden_states.reshape(s, h)
    y = _moe(x, gate_kernel, e_bias.reshape(1, E), expert_gate, expert_up,
             expert_down, sh_gate, sh_up, sh_down)
    return y.reshape(b, s, h)
